# baseline (speedup 1.0000x reference)
import jax
import jax.numpy as jnp
from jax.experimental import pallas as pl
from jax.experimental.pallas import tpu as pltpu

_SCALE = 1.0 / 0.6


def _pick_block(e: int) -> int:
    for cand in (6400, 3200, 2560, 2000, 1600, 1280, 800, 640, 400, 320, 160, 80, 40, 8):
        if e % cand == 0:
            return cand
    return e


def _fused_kernel(src_ref, emb_ref, rbf_ref, dvec_ref, batch_ref,
                  w1_ref, w2_ref, wrbf_ref, wout_ref,
                  out_ref, acc_ref, s_ref, starts_ref, ends_ref):
    i = pl.program_id(0)
    nsteps = pl.num_programs(0)
    bsz = starts_ref.shape[1]

    @pl.when(i == 0)
    def _init():
        acc_ref[:] = jnp.zeros_like(acc_ref)
        s_ref[:] = jnp.zeros_like(s_ref)
        b_ids = jax.lax.broadcasted_iota(jnp.int32, (bsz, 1), 0)
        lt = (batch_ref[:] < b_ids).astype(jnp.int32)
        le = (batch_ref[:] <= b_ids).astype(jnp.int32)
        starts_ref[0, :] = jnp.sum(lt, axis=1)
        ends_ref[0, :] = jnp.sum(le, axis=1)

    s = s_ref[:]
    d = dvec_ref[:]
    nsq = jnp.dot(d * d, jnp.ones((3, 1), jnp.float32),
                  preferred_element_type=jnp.float32)
    w = s * jax.lax.rsqrt(nsq)
    dw = d * w

    src = src_ref[:]
    onehot = ((src >= starts_ref[:]).astype(jnp.float32)
              - (src >= ends_ref[:]).astype(jnp.float32))

    d4 = jnp.concatenate([d, jnp.ones_like(s)], axis=1)
    left = jnp.concatenate(
        [onehot * dw[:, 0:1], onehot * dw[:, 1:2], onehot * dw[:, 2:3],
         onehot], axis=1)
    flag = jnp.where(i > 0, 1.0, 0.0).astype(jnp.float32)
    acc_ref[:] += flag * jax.lax.dot_general(
        left, d4, dimension_numbers=(((0,), (0,)), ((), ())),
        preferred_element_type=jnp.float32)

    h = jnp.dot(emb_ref[:], w1_ref[:], preferred_element_type=jnp.float32)
    h = jax.nn.silu(h)
    h = jnp.dot(h, w2_ref[:], preferred_element_type=jnp.float32)
    r = jnp.dot(rbf_ref[:], wrbf_ref[:], preferred_element_type=jnp.float32)
    s_ref[:] = jnp.dot(h * r, wout_ref[:],
                       preferred_element_type=jnp.float32)

    @pl.when(i == nsteps - 1)
    def _fin():
        a = acc_ref[:]
        cnt = a[bsz * 3:bsz * 4, 3:4]
        lat = jnp.concatenate(
            [a[0:bsz, 0:3], a[bsz:2 * bsz, 0:3], a[2 * bsz:3 * bsz, 0:3]],
            axis=1)
        out_ref[:] = jnp.where(cnt > 0, lat / cnt, 0.0)


def kernel(edge_emb, edge_index, distance_vec, lattice, batch, rbf, W1, W2, W_rbf, W_out):
    e, d_dim = edge_emb.shape
    n = batch.shape[0]
    b = lattice.shape[0]
    r_dim = rbf.shape[1]
    blk = _pick_block(e)
    nb = e // blk
    grid = (nb + 1,)

    src = edge_index[0].astype(jnp.int32).reshape(e, 1)
    batch2d = batch.astype(jnp.int32).reshape(1, n)
    wout_col = W_out.astype(jnp.float32).reshape(d_dim, 1)
    w2_scaled = W2 * jnp.float32(_SCALE)

    def cur_map(i):
        return (jnp.minimum(i, nb - 1), 0)

    def prev_map(i):
        return (jnp.maximum(i - 1, 0), 0)

    out = pl.pallas_call(
        _fused_kernel,
        grid=grid,
        in_specs=[
            pl.BlockSpec((blk, 1), prev_map),
            pl.BlockSpec((blk, d_dim), cur_map),
            pl.BlockSpec((blk, r_dim), cur_map),
            pl.BlockSpec((blk, 3), prev_map),
            pl.BlockSpec((1, n), lambda i: (0, 0)),
            pl.BlockSpec((d_dim, d_dim), lambda i: (0, 0)),
            pl.BlockSpec((d_dim, d_dim), lambda i: (0, 0)),
            pl.BlockSpec((r_dim, d_dim), lambda i: (0, 0)),
            pl.BlockSpec((d_dim, 1), lambda i: (0, 0)),
        ],
        out_specs=pl.BlockSpec((b, 9), lambda i: (0, 0)),
        out_shape=jax.ShapeDtypeStruct((b, 9), jnp.float32),
        scratch_shapes=[
            pltpu.VMEM((4 * b, 4), jnp.float32),
            pltpu.VMEM((blk, 1), jnp.float32),
            pltpu.VMEM((1, b), jnp.int32),
            pltpu.VMEM((1, b), jnp.int32),
        ],
    )(src, edge_emb, rbf, distance_vec, batch2d, W1, w2_scaled, W_rbf, wout_col)

    lat = out.reshape(b, 3, 3)
    return 0.5 * (lat + jnp.swapaxes(lat, 1, 2))


# speedup vs baseline: 1.3017x; 1.3017x over previous
import functools

import jax
import jax.numpy as jnp
from jax import lax
from jax.experimental import pallas as pl
from jax.experimental.pallas import tpu as pltpu
from jax.experimental.pallas import tpu_sc as plsc

_SCALE = 1.0 / 0.6


def _pick_block(e: int) -> int:
    for cand in (3200, 2560, 2000, 1600, 1280, 800, 640, 400, 320, 160, 80, 40, 8):
        if e % cand == 0:
            return cand
    return e


def _score_kernel(emb_ref, rbf_ref, dvec_ref, w1_ref, w2_ref, wrbf_ref,
                  wout_ref, w_out_ref):
    h = jnp.dot(emb_ref[:], w1_ref[:], preferred_element_type=jnp.float32)
    h = jax.nn.silu(h)
    h = jnp.dot(h, w2_ref[:], preferred_element_type=jnp.float32)
    r = jnp.dot(rbf_ref[:], wrbf_ref[:], preferred_element_type=jnp.float32)
    s = jnp.dot(h * r, wout_ref[:], preferred_element_type=jnp.float32)
    d = dvec_ref[:]
    nsq = jnp.dot(d * d, jnp.ones((3, 1), jnp.float32),
                  preferred_element_type=jnp.float32)
    w_out_ref[:] = s * jax.lax.rsqrt(nsq)


def _make_sc_reduce(e: int, n: int, bsz: int, nw: int, chunk: int):
    e_per_w = e // nw
    nchunks = e_per_w // chunk
    nvec = chunk // 16
    nslots = 16 * bsz
    npay = 10

    mesh = plsc.VectorSubcoreMesh(core_axis_name="c", subcore_axis_name="s")

    @functools.partial(
        pl.kernel, mesh=mesh,
        out_type=jax.ShapeDtypeStruct((nw, npay * nslots), jnp.float32),
        scratch_types=[
            pltpu.VMEM((n,), jnp.int32),
            pltpu.VMEM((chunk,), jnp.int32),
            pltpu.VMEM((chunk,), jnp.float32),
            pltpu.VMEM((chunk,), jnp.float32),
            pltpu.VMEM((chunk,), jnp.float32),
            pltpu.VMEM((chunk,), jnp.float32),
            pltpu.VMEM((npay * nslots,), jnp.float32),
        ],
        compiler_params=pltpu.CompilerParams(needs_layout_passes=False),
    )
    def sc_reduce(src_hbm, w_hbm, d0_hbm, d1_hbm, d2_hbm, batch_hbm, out_hbm,
                  batch_v, src_v, w_v, d0_v, d1_v, d2_v, acc_v):
        wid = lax.axis_index("s") * 2 + lax.axis_index("c")
        base = wid * e_per_w
        pltpu.sync_copy(batch_hbm, batch_v)

        zeros16 = jnp.zeros((16,), jnp.float32)
        ones16 = jnp.ones((16,), jnp.float32)
        iota16 = lax.iota(jnp.int32, 16)

        def _zero(j, carry):
            acc_v[pl.ds(j * 16, 16)] = zeros16
            return carry
        lax.fori_loop(0, (npay * nslots) // 16, _zero, 0)

        def _chunk(c, carry):
            off = base + c * chunk
            pltpu.sync_copy(src_hbm.at[pl.ds(off, chunk)], src_v)
            pltpu.sync_copy(w_hbm.at[pl.ds(off, chunk)], w_v)
            pltpu.sync_copy(d0_hbm.at[pl.ds(off, chunk)], d0_v)
            pltpu.sync_copy(d1_hbm.at[pl.ds(off, chunk)], d1_v)
            pltpu.sync_copy(d2_hbm.at[pl.ds(off, chunk)], d2_v)

            def _vec(j, carry2):
                sl = pl.ds(j * 16, 16)
                srcj = src_v[sl]
                wj = w_v[sl]
                a0 = d0_v[sl]
                a1 = d1_v[sl]
                a2 = d2_v[sl]
                g = plsc.load_gather(batch_v, [srcj])
                gl = g * 16 + iota16
                dw0 = a0 * wj
                dw1 = a1 * wj
                dw2 = a2 * wj
                payloads = (dw0 * a0, dw0 * a1, dw0 * a2,
                            dw1 * a0, dw1 * a1, dw1 * a2,
                            dw2 * a0, dw2 * a1, dw2 * a2,
                            ones16)
                for k, v in enumerate(payloads):
                    plsc.addupdate_scatter(acc_v, [gl + k * nslots], v)
                return carry2
            lax.fori_loop(0, nvec, _vec, 0)
            return carry
        lax.fori_loop(0, nchunks, _chunk, 0)

        pltpu.sync_copy(acc_v, out_hbm.at[wid])

    return sc_reduce


def kernel(edge_emb, edge_index, distance_vec, lattice, batch, rbf, W1, W2, W_rbf, W_out):
    e, d_dim = edge_emb.shape
    n = batch.shape[0]
    bsz = lattice.shape[0]
    r_dim = rbf.shape[1]
    blk = _pick_block(e)
    grid = (e // blk,)

    batch_i32 = batch.astype(jnp.int32)
    src_i32 = edge_index[0].astype(jnp.int32)
    wout_col = W_out.astype(jnp.float32).reshape(d_dim, 1)
    w2_scaled = W2 * jnp.float32(_SCALE)
    dt = distance_vec.T

    w_edge = pl.pallas_call(
        _score_kernel,
        grid=grid,
        in_specs=[
            pl.BlockSpec((blk, d_dim), lambda i: (i, 0)),
            pl.BlockSpec((blk, r_dim), lambda i: (i, 0)),
            pl.BlockSpec((blk, 3), lambda i: (i, 0)),
            pl.BlockSpec((d_dim, d_dim), lambda i: (0, 0)),
            pl.BlockSpec((d_dim, d_dim), lambda i: (0, 0)),
            pl.BlockSpec((r_dim, d_dim), lambda i: (0, 0)),
            pl.BlockSpec((d_dim, 1), lambda i: (0, 0)),
        ],
        out_specs=pl.BlockSpec((blk, 1), lambda i: (i, 0)),
        out_shape=jax.ShapeDtypeStruct((e, 1), jnp.float32),
    )(edge_emb, rbf, distance_vec, W1, w2_scaled, W_rbf, wout_col)

    sc_reduce = _make_sc_reduce(e, n, bsz, nw=32, chunk=2000)
    partials = sc_reduce(src_i32, w_edge.reshape(e), dt[0], dt[1], dt[2],
                         batch_i32)

    res = partials.sum(axis=0).reshape(10, bsz, 16).sum(-1)
    cnt = res[9]
    lat = jnp.where(cnt > 0, res[:9] / cnt, 0.0).T.reshape(bsz, 3, 3)
    return 0.5 * (lat + jnp.swapaxes(lat, 1, 2))


# speedup vs baseline: 1.3129x; 1.0086x over previous
import functools

import jax
import jax.numpy as jnp
from jax import lax
from jax.experimental import pallas as pl
from jax.experimental.pallas import tpu as pltpu
from jax.experimental.pallas import tpu_sc as plsc

_SCALE = 1.0 / 0.6


def _pick_block(e: int) -> int:
    for cand in (3200, 2560, 2000, 1600, 1280, 800, 640, 400, 320, 160, 80, 40, 8):
        if e % cand == 0:
            return cand
    return e


def _score_kernel(emb_ref, rbf_ref, dvec_ref, w1_ref, w2_ref, wrbf_ref,
                  wout_ref, w_out_ref):
    h = jnp.dot(emb_ref[:], w1_ref[:], preferred_element_type=jnp.float32)
    h = jax.nn.silu(h) * _SCALE
    h = jnp.dot(h, w2_ref[:], preferred_element_type=jnp.float32)
    r = jnp.dot(rbf_ref[:], wrbf_ref[:], preferred_element_type=jnp.float32)
    s = jnp.dot(h * r, wout_ref[:], preferred_element_type=jnp.float32)
    d = dvec_ref[:]
    nsq = jnp.sum(d * d, axis=1, keepdims=True)
    w_out_ref[:] = s / jnp.sqrt(nsq)


def _make_sc_reduce(e: int, n: int, bsz: int, nw: int, chunk: int):
    e_per_w = e // nw
    nchunks = e_per_w // chunk
    nvec = chunk // 16
    nslots = 16 * bsz
    npay = 10

    mesh = plsc.VectorSubcoreMesh(core_axis_name="c", subcore_axis_name="s")

    @functools.partial(
        pl.kernel, mesh=mesh,
        out_type=jax.ShapeDtypeStruct((nw, npay * nslots), jnp.float32),
        scratch_types=[
            pltpu.VMEM((n,), jnp.int32),
            pltpu.VMEM((chunk,), jnp.int32),
            pltpu.VMEM((chunk,), jnp.float32),
            pltpu.VMEM((chunk,), jnp.float32),
            pltpu.VMEM((chunk,), jnp.float32),
            pltpu.VMEM((chunk,), jnp.float32),
            pltpu.VMEM((npay * nslots,), jnp.float32),
        ],
        compiler_params=pltpu.CompilerParams(needs_layout_passes=False),
    )
    def sc_reduce(src_hbm, w_hbm, d0_hbm, d1_hbm, d2_hbm, batch_hbm, out_hbm,
                  batch_v, src_v, w_v, d0_v, d1_v, d2_v, acc_v):
        wid = lax.axis_index("s") * 2 + lax.axis_index("c")
        base = wid * e_per_w
        pltpu.sync_copy(batch_hbm, batch_v)

        zeros16 = jnp.zeros((16,), jnp.float32)
        ones16 = jnp.ones((16,), jnp.float32)
        iota16 = lax.iota(jnp.int32, 16)

        def _zero(j, carry):
            acc_v[pl.ds(j * 16, 16)] = zeros16
            return carry
        lax.fori_loop(0, (npay * nslots) // 16, _zero, 0)

        def _chunk(c, carry):
            off = base + c * chunk
            pltpu.sync_copy(src_hbm.at[pl.ds(off, chunk)], src_v)
            pltpu.sync_copy(w_hbm.at[pl.ds(off, chunk)], w_v)
            pltpu.sync_copy(d0_hbm.at[pl.ds(off, chunk)], d0_v)
            pltpu.sync_copy(d1_hbm.at[pl.ds(off, chunk)], d1_v)
            pltpu.sync_copy(d2_hbm.at[pl.ds(off, chunk)], d2_v)

            def _vec(j, carry2):
                sl = pl.ds(j * 16, 16)
                srcj = src_v[sl]
                wj = w_v[sl]
                a0 = d0_v[sl]
                a1 = d1_v[sl]
                a2 = d2_v[sl]
                g = plsc.load_gather(batch_v, [srcj])
                gl = g * 16 + iota16
                dw0 = a0 * wj
                dw1 = a1 * wj
                dw2 = a2 * wj
                payloads = (dw0 * a0, dw0 * a1, dw0 * a2,
                            dw1 * a0, dw1 * a1, dw1 * a2,
                            dw2 * a0, dw2 * a1, dw2 * a2,
                            ones16)
                for k, v in enumerate(payloads):
                    plsc.addupdate_scatter(acc_v, [gl + k * nslots], v)
                return carry2
            lax.fori_loop(0, nvec, _vec, 0)
            return carry
        lax.fori_loop(0, nchunks, _chunk, 0)

        pltpu.sync_copy(acc_v, out_hbm.at[wid])

    return sc_reduce


def kernel(edge_emb, edge_index, distance_vec, lattice, batch, rbf, W1, W2, W_rbf, W_out):
    e, d_dim = edge_emb.shape
    n = batch.shape[0]
    bsz = lattice.shape[0]
    r_dim = rbf.shape[1]
    blk = _pick_block(e)
    grid = (e // blk,)

    batch_i32 = batch.astype(jnp.int32)
    src_i32 = edge_index[0].astype(jnp.int32)
    wout_col = W_out.astype(jnp.float32).reshape(d_dim, 1)
    dt = distance_vec.T

    w_edge = pl.pallas_call(
        _score_kernel,
        grid=grid,
        in_specs=[
            pl.BlockSpec((blk, d_dim), lambda i: (i, 0)),
            pl.BlockSpec((blk, r_dim), lambda i: (i, 0)),
            pl.BlockSpec((blk, 3), lambda i: (i, 0)),
            pl.BlockSpec((d_dim, d_dim), lambda i: (0, 0)),
            pl.BlockSpec((d_dim, d_dim), lambda i: (0, 0)),
            pl.BlockSpec((r_dim, d_dim), lambda i: (0, 0)),
            pl.BlockSpec((d_dim, 1), lambda i: (0, 0)),
        ],
        out_specs=pl.BlockSpec((blk, 1), lambda i: (i, 0)),
        out_shape=jax.ShapeDtypeStruct((e, 1), jnp.float32),
    )(edge_emb, rbf, distance_vec, W1, W2, W_rbf, wout_col)

    sc_reduce = _make_sc_reduce(e, n, bsz, nw=32, chunk=2000)
    partials = sc_reduce(src_i32, w_edge.reshape(e), dt[0], dt[1], dt[2],
                         batch_i32)

    res = partials.sum(axis=0).reshape(10, bsz, 16).sum(-1)
    cnt = res[9]
    lat = jnp.where(cnt > 0, res[:9] / cnt, 0.0).T.reshape(bsz, 3, 3)
    return 0.5 * (lat + jnp.swapaxes(lat, 1, 2))
